# baseline (speedup 1.0000x reference)
import functools

import jax
import jax.numpy as jnp
from jax import lax
from jax.experimental import pallas as pl
from jax.experimental.pallas import tpu as pltpu
from jax.experimental.pallas import tpu_sc as plsc

NUM_CORES = 2
NUM_SUBCORES = 16
NUM_WORKERS = NUM_CORES * NUM_SUBCORES


@functools.cache
def _make_gather(hist: int, batch: int, dim: int):
    assert batch % (NUM_WORKERS * 128) == 0 and dim == 32 and hist % 2 == 0
    bw = batch // NUM_WORKERS
    nct = bw // 128
    rbs = dim // 8
    mesh = plsc.VectorSubcoreMesh(core_axis_name="c", subcore_axis_name="s")

    @functools.partial(
        pl.kernel,
        out_type=jax.ShapeDtypeStruct((hist, rbs, batch // 128, 8, 128),
                                      jnp.float32),
        mesh=mesh,
        scratch_types=[
            pltpu.VMEM((hist, bw), jnp.int32),
            pltpu.VMEM((bw, dim), jnp.float32),
            pltpu.VMEM((bw, dim), jnp.float32),
            pltpu.VMEM((rbs, nct, 8, 128), jnp.float32),
            pltpu.VMEM((rbs, nct, 8, 128), jnp.float32),
            pltpu.SemaphoreType.DMA,
            pltpu.SemaphoreType.DMA,
            pltpu.SemaphoreType.DMA,
            pltpu.SemaphoreType.DMA,
        ],
        compiler_params=pltpu.CompilerParams(use_tc_tiling_on_sc=False,
                                             needs_layout_passes=False),
    )
    def gather_kernel(idxt_hbm, table_hbm, out_hbm, idx_v, rows0, rows1,
                      slab0, slab1, gsem0, gsem1, osem0, osem1):
        wid = lax.axis_index("s") * NUM_CORES + lax.axis_index("c")
        b0 = wid * bw
        ct0 = wid * nct
        pltpu.sync_copy(idxt_hbm.at[:, pl.ds(b0, bw)], idx_v)

        def gdesc(h, rbuf, gsem):
            return pltpu.make_async_copy(
                table_hbm.at[idx_v.at[h]], rbuf, gsem)

        def odesc(h, sbuf, osem):
            return pltpu.make_async_copy(
                sbuf, out_hbm.at[h, :, pl.ds(ct0, nct)], osem)

        gdesc(0, rows0, gsem0).start()
        gdesc(1, rows1, gsem1).start()

        @pl.loop(0, hist, step=2)
        def _(i):
            for sub, rbuf, sbuf, gsem, osem in (
                (0, rows0, slab0, gsem0, osem0),
                (1, rows1, slab1, gsem1, osem1),
            ):
                h = i + sub
                gdesc(h, rbuf, gsem).wait()

                @pl.when(h >= 2)
                def _():
                    odesc(h - 2, sbuf, osem).wait()

                for ctoff in range(nct):
                    base = lax.iota(jnp.int32, 16) + ctoff * 128

                    @plsc.parallel_loop(0, 8, unroll=4)
                    def _(bg):
                        bs0 = bg * 16
                        bvec = bs0 + base
                        for rb in range(rbs):
                            for ds in range(8):
                                dvec = jnp.full((16,), rb * 8 + ds,
                                                jnp.int32)
                                vals = plsc.load_gather(rbuf, [bvec, dvec])
                                sbuf[rb, ctoff, ds, pl.ds(bs0, 16)] = vals

                @pl.when(h + 2 < hist)
                def _():
                    gdesc(h + 2, rbuf, gsem).start()

                odesc(h, sbuf, osem).start()

        odesc(hist - 2, slab0, osem0).wait()
        odesc(hist - 1, slab1, osem1).wait()

    return gather_kernel


def kernel(idx, weight):
    batch, hist = idx.shape
    dim = weight.shape[1]
    idxt = jnp.transpose(idx).astype(jnp.int32)
    out5 = _make_gather(hist, batch, dim)(idxt, weight)
    return jnp.reshape(jnp.transpose(out5, (2, 4, 0, 1, 3)),
                       (batch, hist, dim))


# speedup vs baseline: 1.0778x; 1.0778x over previous
import functools

import jax
import jax.numpy as jnp
from jax import lax
from jax.experimental import pallas as pl
from jax.experimental.pallas import tpu as pltpu
from jax.experimental.pallas import tpu_sc as plsc

NUM_CORES = 2
NUM_SUBCORES = 16
NUM_WORKERS = NUM_CORES * NUM_SUBCORES


@functools.cache
def _make_gather(hist: int, batch: int, dim: int):
    assert batch % (NUM_WORKERS * 128) == 0 and dim == 32 and hist % 2 == 0
    bw = batch // NUM_WORKERS
    nct = bw // 128
    rbs = dim // 8
    mesh = plsc.VectorSubcoreMesh(core_axis_name="c", subcore_axis_name="s")

    @functools.partial(
        pl.kernel,
        out_type=jax.ShapeDtypeStruct((hist, rbs, batch // 128, 8, 128),
                                      jnp.float32),
        mesh=mesh,
        scratch_types=[
            pltpu.VMEM((hist, bw), jnp.int32),
            pltpu.VMEM((bw, dim), jnp.float32),
            pltpu.VMEM((bw, dim), jnp.float32),
            pltpu.VMEM((rbs, nct, 8, 128), jnp.float32),
            pltpu.VMEM((rbs, nct, 8, 128), jnp.float32),
            pltpu.SemaphoreType.DMA,
            pltpu.SemaphoreType.DMA,
            pltpu.SemaphoreType.DMA,
            pltpu.SemaphoreType.DMA,
        ],
        compiler_params=pltpu.CompilerParams(use_tc_tiling_on_sc=False,
                                             needs_layout_passes=False),
    )
    def gather_kernel(idxt_hbm, table_hbm, out_hbm, idx_v, rows0, rows1,
                      slab0, slab1, gsem0, gsem1, osem0, osem1):
        wid = lax.axis_index("s") * NUM_CORES + lax.axis_index("c")
        b0 = wid * bw
        ct0 = wid * nct
        pltpu.sync_copy(idxt_hbm.at[:, pl.ds(b0, bw)], idx_v)

        def gdesc(h, rbuf, gsem):
            return pltpu.make_async_copy(
                table_hbm.at[idx_v.at[h]], rbuf, gsem)

        def odesc(h, sbuf, osem):
            return pltpu.make_async_copy(
                sbuf, out_hbm.at[h, :, pl.ds(ct0, nct)], osem)

        gdesc(0, rows0, gsem0).start()
        gdesc(1, rows1, gsem1).start()

        @pl.loop(0, hist, step=2)
        def _(i):
            for sub, rbuf, sbuf, gsem, osem in (
                (0, rows0, slab0, gsem0, osem0),
                (1, rows1, slab1, gsem1, osem1),
            ):
                h = i + sub
                gdesc(h, rbuf, gsem).wait()

                @pl.when(h >= 2)
                def _():
                    odesc(h - 2, sbuf, osem).wait()

                base = lax.iota(jnp.int32, 16)

                @plsc.parallel_loop(0, bw // 16, unroll=2)
                def _(bg):
                    ctoff = bg // 8
                    bs0 = (bg % 8) * 16
                    bvec = bg * 16 + base
                    for rb in range(rbs):
                        for ds in range(8):
                            dvec = jnp.full((16,), rb * 8 + ds,
                                            jnp.int32)
                            vals = plsc.load_gather(rbuf, [bvec, dvec])
                            sbuf[rb, ctoff, ds, pl.ds(bs0, 16)] = vals

                @pl.when(h + 2 < hist)
                def _():
                    gdesc(h + 2, rbuf, gsem).start()

                odesc(h, sbuf, osem).start()

        odesc(hist - 2, slab0, osem0).wait()
        odesc(hist - 1, slab1, osem1).wait()

    return gather_kernel


def kernel(idx, weight):
    batch, hist = idx.shape
    dim = weight.shape[1]
    idxt = jnp.transpose(idx).astype(jnp.int32)
    out5 = _make_gather(hist, batch, dim)(idxt, weight)
    return jnp.reshape(jnp.transpose(out5, (2, 4, 0, 1, 3)),
                       (batch, hist, dim))
